# baseline (speedup 1.0000x reference)
import functools

import jax
import jax.numpy as jnp
from jax import lax
from jax.experimental import pallas as pl
from jax.experimental.pallas import tpu as pltpu
from jax.experimental.pallas import tpu_sc as plsc

EMBED = 256
NH = 8
NL = 4
NP = 8
DH = 32
_SS = ((92, 160), (46, 80), (23, 40), (12, 20))
NV = 19560
_PLVL = []
_PBASE = []
_acc = 0
for _h, _w in _SS:
    _PBASE.append(_acc)
    _PLVL.append((_h + 1) * (_w + 1))
    _acc += (_h + 1) * (_w + 1)
PTOT = _acc

BS = 2
NQ = 2048
ROWS = BS * NQ * NH
PPR = NL * NP
WPR = PPR * 4
NWORK = 32
RPW = ROWS // NWORK
BLK = 8
NBLK = RPW // BLK
LANES = 16


def _mm_bias(x, w, b, block_m=512):
    M, K = x.shape
    N = w.shape[1]
    Mp = ((M + block_m - 1) // block_m) * block_m
    xp = jnp.pad(x, ((0, Mp - M), (0, 0))) if Mp != M else x

    def body(x_ref, w_ref, b_ref, o_ref):
        o_ref[...] = jnp.dot(x_ref[...], w_ref[...],
                             preferred_element_type=jnp.float32) + b_ref[...]

    out = pl.pallas_call(
        body,
        grid=(Mp // block_m,),
        in_specs=[pl.BlockSpec((block_m, K), lambda i: (i, 0)),
                  pl.BlockSpec((K, N), lambda i: (0, 0)),
                  pl.BlockSpec((1, N), lambda i: (0, 0))],
        out_specs=pl.BlockSpec((block_m, N), lambda i: (i, 0)),
        out_shape=jax.ShapeDtypeStruct((Mp, N), jnp.float32),
    )(xp, w, b[None, :])
    return out[:M]


def _sc_gather_reduce(vt4, idx, wts):
    mesh = plsc.VectorSubcoreMesh(core_axis_name="c", subcore_axis_name="s")

    @functools.partial(
        pl.kernel,
        out_type=jax.ShapeDtypeStruct((ROWS * DH,), jnp.float32),
        mesh=mesh,
        scratch_types=[
            pltpu.VMEM((BLK * PPR,), jnp.int32),
            pltpu.VMEM((BLK * WPR,), jnp.float32),
            pltpu.VMEM((BLK, PPR, 128), jnp.float32),
            pltpu.VMEM((BLK * DH,), jnp.float32),
            pltpu.SemaphoreType.DMA,
        ],
    )
    def body(vt_hbm, idx_hbm, w_hbm, out_hbm, idx_v, w_v, rows_v, outb, sem):
        wid = lax.axis_index("s") * 2 + lax.axis_index("c")
        base = wid * RPW
        splats = [jnp.full((LANES, 1), t, jnp.int32) for t in range(LANES)]
        gdn = lax.GatherDimensionNumbers(
            offset_dims=(), collapsed_slice_dims=(0,), start_index_map=(0,))

        def bcast(vec, t):
            return lax.gather(vec, splats[t], gdn, (1,),
                              mode=lax.GatherScatterMode.PROMISE_IN_BOUNDS)

        def blk_body(i, carry):
            r0 = base + i * BLK
            pltpu.sync_copy(idx_hbm.at[pl.ds(r0 * PPR, BLK * PPR)], idx_v)
            pltpu.sync_copy(w_hbm.at[pl.ds(r0 * WPR, BLK * WPR)], w_v)
            copies = [
                pltpu.async_copy(
                    vt_hbm.at[idx_v.at[pl.ds(r * PPR, PPR)]], rows_v.at[r], sem)
                for r in range(BLK)
            ]
            for cp in copies:
                cp.wait()

            def row_body(r, carry2):
                def chunk(cc, acc):
                    a0, a1 = acc
                    wchunk = w_v[pl.ds(r * WPR + cc * LANES, LANES)]
                    for t in range(LANES):
                        p = cc * 4 + t // 4
                        c4 = t % 4
                        wb = bcast(wchunk, t)
                        lo = rows_v[r, p, pl.ds(c4 * DH, LANES)]
                        hi = rows_v[r, p, pl.ds(c4 * DH + LANES, LANES)]
                        a0 = a0 + wb * lo
                        a1 = a1 + wb * hi
                    return a0, a1

                z = jnp.zeros((LANES,), jnp.float32)
                a0, a1 = lax.fori_loop(0, WPR // LANES, chunk, (z, z))
                outb[pl.ds(r * DH, LANES)] = a0
                outb[pl.ds(r * DH + LANES, LANES)] = a1
                return carry2

            lax.fori_loop(0, BLK, row_body, 0)
            pltpu.sync_copy(outb, out_hbm.at[pl.ds(r0 * DH, BLK * DH)])
            return carry

        lax.fori_loop(0, NBLK, blk_body, 0)

    return body(vt4, idx, wts)


def kernel(query, value, reference_points, spatial_shapes, level_start_index,
           W_off, b_off, W_attn, b_attn, W_val, b_val, W_out, b_out):
    bs, nq, d = query.shape
    nv = value.shape[1]

    v = _mm_bias(value.reshape(bs * nv, d), W_val, b_val)
    v4 = v.reshape(bs, nv, NH, DH)
    tables = []
    start = 0
    for (H, W) in _SS:
        vl = v4[:, start:start + H * W].reshape(bs, H, W, NH, DH)
        start += H * W
        P = jnp.pad(vl, ((0, 0), (1, 1), (1, 1), (0, 0), (0, 0)))
        quad = jnp.concatenate(
            [P[:, :-1, :-1], P[:, :-1, 1:], P[:, 1:, :-1], P[:, 1:, 1:]],
            axis=-1)
        tables.append(
            quad.transpose(0, 3, 1, 2, 4).reshape(bs, NH, (H + 1) * (W + 1), 4 * DH))
    vt4 = jnp.concatenate(tables, axis=2).reshape(bs * NH * PTOT, 4 * DH)

    qw = jnp.concatenate([W_off, W_attn], axis=1)
    qb = jnp.concatenate([b_off, b_attn], axis=0)
    qproj = _mm_bias(query.reshape(bs * nq, d), qw, qb)
    off = qproj[:, :NH * NL * NP * 2].reshape(bs, nq, NH, NL, NP, 2)
    aw = jax.nn.softmax(
        qproj[:, NH * NL * NP * 2:].reshape(bs, nq, NH, NL * NP), axis=-1)
    aw = aw.reshape(bs, nq, NH, NL, NP)

    ss_f = spatial_shapes.astype(jnp.float32)
    norm = jnp.stack([ss_f[:, 1], ss_f[:, 0]], axis=-1)
    nZ = reference_points.shape[2]
    ref = reference_points[:, :, None, None, None, :, :]
    off_n = off / norm[None, None, None, :, None, :]
    off_n = off_n.reshape(bs, nq, NH, NL, NP // nZ, nZ, 2)
    loc = (ref + off_n).reshape(bs, nq, NH, NL, NP, 2)

    Wl = jnp.array([s[1] for s in _SS], jnp.float32)[:, None]
    Hl = jnp.array([s[0] for s in _SS], jnp.float32)[:, None]
    x = loc[..., 0] * Wl - 0.5
    y = loc[..., 1] * Hl - 0.5
    x0 = jnp.floor(x)
    y0 = jnp.floor(y)
    tx = x - x0
    ty = y - y0
    xi = jnp.clip(x0, -1.0, Wl - 1.0).astype(jnp.int32)
    yi = jnp.clip(y0, -1.0, Hl - 1.0).astype(jnp.int32)
    Wli = jnp.array([s[1] for s in _SS], jnp.int32)[:, None]
    pbase = jnp.array(_PBASE, jnp.int32)[:, None]
    bh = ((jnp.arange(bs, dtype=jnp.int32)[:, None] * NH
           + jnp.arange(NH, dtype=jnp.int32)[None, :]) * PTOT)
    bh = bh[:, None, :, None, None]
    pidx = bh + pbase + (yi + 1) * (Wli + 1) + (xi + 1)

    vx0 = ((x0 >= 0) & (x0 < Wl)).astype(jnp.float32)
    vx1 = ((x0 + 1 >= 0) & (x0 + 1 < Wl)).astype(jnp.float32)
    vy0 = ((y0 >= 0) & (y0 < Hl)).astype(jnp.float32)
    vy1 = ((y0 + 1 >= 0) & (y0 + 1 < Hl)).astype(jnp.float32)
    wx0 = (1.0 - tx) * vx0
    wx1 = tx * vx1
    wy0 = (1.0 - ty) * vy0
    wy1 = ty * vy1
    w4 = jnp.stack([wx0 * wy0, wx1 * wy0, wx0 * wy1, wx1 * wy1], axis=-1)
    w4 = w4 * aw[..., None]

    idx = pidx.reshape(ROWS * PPR)
    wts = w4.reshape(ROWS * WPR)

    res = _sc_gather_reduce(vt4, idx, wts)

    out = _mm_bias(res.reshape(bs * nq, d), W_out, b_out)
    return out.reshape(bs, nq, d)
